# Initial kernel scaffold; baseline (speedup 1.0000x reference)
import jax
import jax.numpy as jnp
from jax.experimental import pallas as pl


def kernel(QT, SLI, SST, SOLIN, layer_mass, eta, W_base, b_base, coef_qt, int_qt, coef_sli, int_sli):
    raise NotImplementedError("write your pallas kernel here")


# speedup vs baseline: 5.2385x; 5.2385x over previous
import jax
import jax.numpy as jnp
from jax.experimental import pallas as pl

NZ = 34
E = 8
MAX_QT = 15
MAX_SLI = 18
SCALE = 1.0
EP = 72
BT = 512


def _routed_kernel(x_ref, eta_ref, gw_ref, gb_ref, out_ref):
    x = x_ref[...]
    y = jax.lax.dot_general(
        gw_ref[...], x, (((1,), (0,)), ((), ())),
        preferred_element_type=jnp.float32)
    y = y + gb_ref[...]
    eta = eta_ref[...]
    acc = jnp.zeros((EP, y.shape[1]), jnp.float32)
    for e in range(E):
        acc = acc + jnp.where(eta == e, y[e * EP:(e + 1) * EP, :], 0.0)
    out_ref[...] = acc[:2 * NZ, :]


def kernel(QT, SLI, SST, SOLIN, layer_mass, eta, W_base, b_base,
           coef_qt, int_qt, coef_sli, int_sli):
    nz, h, w = QT.shape
    n = h * w
    X = jnp.concatenate([
        QT.reshape(nz, n), SLI.reshape(nz, n),
        SST.reshape(1, n), SOLIN.reshape(1, n)], axis=0)

    coef_cat = jnp.concatenate([coef_qt, coef_sli], axis=1)
    int_cat = jnp.concatenate([int_qt, int_sli], axis=1)
    npred = MAX_QT + MAX_SLI
    A = coef_cat[:, :, :npred]
    W_sel = jnp.concatenate([W_base[:MAX_QT], W_base[nz:nz + MAX_SLI]], axis=0)
    b_sel = jnp.concatenate([b_base[:MAX_QT], b_base[nz:nz + MAX_SLI]], axis=0)
    D = jnp.concatenate([
        coef_cat[:, :, npred + 1:npred + 1 + nz],
        coef_cat[:, :, npred + 1 + nz:npred + 1 + 2 * nz],
        coef_cat[:, :, npred:npred + 1],
        coef_cat[:, :, npred + 1 + 2 * nz:],
    ], axis=2)
    G = W_base[None] + SCALE * (jnp.einsum('eoc,cf->eof', A, W_sel) + D)
    g = b_base[None] + SCALE * (jnp.einsum('eoc,c->eo', A, b_sel) + int_cat)

    Gp = jnp.zeros((E, EP, 70), jnp.float32).at[:, :2 * nz].set(G)
    Gp = Gp.reshape(E * EP, 70)
    gp = jnp.zeros((E, EP), jnp.float32).at[:, :2 * nz].set(g)
    gp = gp.reshape(E * EP, 1)
    eta2 = eta.reshape(1, n).astype(jnp.int32)

    out = pl.pallas_call(
        _routed_kernel,
        grid=(n // BT,),
        in_specs=[
            pl.BlockSpec((2 * nz + 2, BT), lambda i: (0, i)),
            pl.BlockSpec((1, BT), lambda i: (0, i)),
            pl.BlockSpec((E * EP, 2 * nz + 2), lambda i: (0, 0)),
            pl.BlockSpec((E * EP, 1), lambda i: (0, 0)),
        ],
        out_specs=pl.BlockSpec((2 * nz, BT), lambda i: (0, i)),
        out_shape=jax.ShapeDtypeStruct((2 * nz, n), jnp.float32),
    )(X, eta2, Gp, gp)
    return out.reshape(2, nz, h, w)
